# Initial kernel scaffold; baseline (speedup 1.0000x reference)
import jax
import jax.numpy as jnp
from jax.experimental import pallas as pl


def kernel(home_team_id, away_team_id, game_features, table, W1, b1, W2, b2, W3, b3, W4, b4):
    raise NotImplementedError("write your pallas kernel here")


# speedup vs baseline: 1.2191x; 1.2191x over previous
import functools

import jax
import jax.numpy as jnp
from jax import lax
from jax.experimental import pallas as pl
from jax.experimental.pallas import tpu as pltpu
from jax.experimental.pallas import tpu_sc as plsc

NUM_TEAMS = 1000000
EMBED_DIM = 16
NUM_FEATURES = 22
BATCH = 16384

NC, NS = 2, 16
NW = NC * NS
CHUNK = 128
B2 = 2 * BATCH
ROWS_PER_W = B2 // NW
K = ROWS_PER_W // CHUNK
NROW = B2 // CHUNK

_sc_mesh = plsc.VectorSubcoreMesh(
    core_axis_name="c", subcore_axis_name="s", num_cores=NC, num_subcores=NS
)


def _gather_body(table_hbm, idx_hbm, out_hbm, idx_v, rows_v, sem):
  wid = lax.axis_index("s") * NC + lax.axis_index("c")
  base = wid * K
  pltpu.sync_copy(idx_hbm.at[pl.ds(base, K)], idx_v)
  copies = [
      pltpu.make_async_copy(table_hbm.at[idx_v.at[j]], rows_v.at[j], sem)
      for j in range(K)
  ]
  for c in copies:
    c.start()
  for c in copies:
    c.wait()
  pltpu.sync_copy(rows_v, out_hbm.at[pl.ds(base, K)])


_gather = pl.kernel(
    _gather_body,
    out_type=jax.ShapeDtypeStruct((NROW, CHUNK, EMBED_DIM), jnp.float32),
    mesh=_sc_mesh,
    scratch_types=[
        pltpu.VMEM((K, CHUNK), jnp.int32),
        pltpu.VMEM((K, CHUNK, EMBED_DIM), jnp.float32),
        pltpu.SemaphoreType.DMA,
    ],
    compiler_params=pltpu.CompilerParams(use_tc_tiling_on_sc=False),
)

BM = 2048
NB = BATCH // BM


def _mlp_body(gf, home, away, w1, b1, w2, b2, w3h, w3a, w3f, b3, w4, b4, out):
  h = jnp.maximum(
      jnp.dot(gf[:], w1[:], preferred_element_type=jnp.float32) + b1[:], 0.0
  )
  fo = jnp.dot(h, w2[:], preferred_element_type=jnp.float32) + b2[:]
  pre = (
      jnp.dot(home[:], w3h[:], preferred_element_type=jnp.float32)
      + jnp.dot(away[:], w3a[:], preferred_element_type=jnp.float32)
      + jnp.dot(fo, w3f[:], preferred_element_type=jnp.float32)
      + b3[:]
  )
  g = jnp.maximum(pre, 0.0)
  out[:] = jnp.dot(g, w4[:], preferred_element_type=jnp.float32) + b4[:]


def _full(shape):
  return pl.BlockSpec(shape, lambda i: (0,) * len(shape))


_mlp = pl.pallas_call(
    _mlp_body,
    grid=(NB,),
    in_specs=[
        pl.BlockSpec((BM, NUM_FEATURES), lambda i: (i, 0)),
        pl.BlockSpec((BM, EMBED_DIM), lambda i: (i, 0)),
        pl.BlockSpec((BM, EMBED_DIM), lambda i: (i + NB, 0)),
        _full((NUM_FEATURES, 16)),
        _full((1, 16)),
        _full((16, 8)),
        _full((1, 8)),
        _full((EMBED_DIM, 8)),
        _full((EMBED_DIM, 8)),
        _full((8, 8)),
        _full((1, 8)),
        _full((8, 1)),
        _full((1, 1)),
    ],
    out_specs=pl.BlockSpec((BM, 1), lambda i: (i, 0)),
    out_shape=jax.ShapeDtypeStruct((BATCH, 1), jnp.float32),
)


@jax.jit
def kernel(home_team_id, away_team_id, game_features, table,
           W1, b1, W2, b2, W3, b3, W4, b4):
  idx = jnp.concatenate(
      [home_team_id.astype(jnp.int32), away_team_id.astype(jnp.int32)]
  ).reshape(NROW, CHUNK)
  emb = _gather(table, idx).reshape(B2, EMBED_DIM)
  return _mlp(
      game_features,
      emb,
      emb,
      W1,
      b1.reshape(1, 16),
      W2,
      b2.reshape(1, 8),
      W3[:EMBED_DIM],
      W3[EMBED_DIM : 2 * EMBED_DIM],
      W3[2 * EMBED_DIM :],
      b3.reshape(1, 8),
      W4,
      b4.reshape(1, 1),
  )
